# baseline (speedup 1.0000x reference)
import functools

import jax
import jax.numpy as jnp
from jax import lax
from jax.experimental import pallas as pl
from jax.experimental.pallas import tpu as pltpu
from jax.experimental.pallas import tpu_sc as plsc

U, I, D, B = 10000, 10000, 16, 4096
BM = 400
NB = U // BM
_TC_PARAMS = pltpu.CompilerParams(vmem_limit_bytes=64 * 1024 * 1024)

_f32 = jnp.float32


def _pass1_body(at_ref, u0f_ref, i0_ref, i1_ref, u1t_ref):
    j = pl.program_id(0)
    at = at_ref[...]
    i1_ref[...] = jnp.dot(at, u0f_ref[...], preferred_element_type=_f32)
    part_t = lax.dot_general(i0_ref[...], at, (((0,), (0,)), ((), ())),
                             preferred_element_type=_f32)

    @pl.when(j == 0)
    def _init():
        u1t_ref[...] = part_t

    @pl.when(j > 0)
    def _acc():
        u1t_ref[...] += part_t


def _pass2_body(a_ref, i1f_ref, u1t_ref, u0_ref, i0f_ref, ufin_ref, ifin_ref,
                i2t_acc):
    j = pl.program_id(0)
    a = a_ref[...]
    u1t_sl = u1t_ref[0]
    u2 = jnp.dot(a, i1f_ref[...], preferred_element_type=_f32)
    ufin_ref[:, 0:D] = (u0_ref[...] + u1t_sl.T + u2) * (1.0 / 3.0)
    part_t = lax.dot_general(u1t_sl, a, (((1,), (0,)), ((), ())),
                             preferred_element_type=_f32)

    @pl.when(j == 0)
    def _init():
        i2t_acc[...] = part_t

    @pl.when(j > 0)
    def _acc():
        i2t_acc[...] += part_t

    @pl.when(j == NB - 1)
    def _fin():
        ifin_ref[:, 0:D] = (i2t_acc[...].T + i0f_ref[...] + i1f_ref[...]) * (1.0 / 3.0)


def _propagate(user_emb, item_emb, A, A_T):
    blk_a = pl.BlockSpec((BM, I), lambda j: (j, 0))
    blk_row = pl.BlockSpec((BM, D), lambda j: (j, 0))
    full_u = pl.BlockSpec((U, D), lambda j: (0, 0))
    full_i = pl.BlockSpec((I, D), lambda j: (0, 0))

    full_ut = pl.BlockSpec((D, U), lambda j: (0, 0))
    i1, u1t = pl.pallas_call(
        _pass1_body,
        grid=(NB,),
        in_specs=[blk_a, full_u, blk_row],
        out_specs=[blk_row, full_ut],
        out_shape=(jax.ShapeDtypeStruct((I, D), _f32),
                   jax.ShapeDtypeStruct((D, U), _f32)),
        compiler_params=_TC_PARAMS,
    )(A_T, user_emb, item_emb)

    blk_row_pad = pl.BlockSpec((BM, 128), lambda j: (j, 0))
    full_i_pad = pl.BlockSpec((I, 128), lambda j: (0, 0))
    u_fin, i_fin = pl.pallas_call(
        _pass2_body,
        grid=(NB,),
        in_specs=[blk_a, full_i,
                  pl.BlockSpec((1, D, BM), lambda j: (j, 0, 0)),
                  blk_row, full_i],
        out_specs=[blk_row_pad, full_i_pad],
        out_shape=(jax.ShapeDtypeStruct((U, 128), _f32),
                   jax.ShapeDtypeStruct((I, 128), _f32)),
        scratch_shapes=[pltpu.VMEM((D, I), _f32)],
        compiler_params=_TC_PARAMS,
    )(A, i1, u1t.reshape(D, NB, BM).swapaxes(0, 1), user_emb, item_emb)
    return u_fin, i_fin


def _make_score_kernel():
    info = plsc.get_sparse_core_info()
    nc, ns = info.num_cores, info.num_subcores
    nw = nc * ns
    bpw = B // nw
    mesh = plsc.VectorSubcoreMesh(core_axis_name="c", subcore_axis_name="s")

    @functools.partial(
        pl.kernel, mesh=mesh,
        out_type=(jax.ShapeDtypeStruct((B,), _f32),
                  jax.ShapeDtypeStruct((B,), _f32)),
        scratch_types=[
            pltpu.VMEM((bpw,), jnp.int32),
            pltpu.VMEM((bpw,), jnp.int32),
            pltpu.VMEM((bpw,), jnp.int32),
            pltpu.VMEM((bpw, 128), _f32),
            pltpu.VMEM((bpw, 128), _f32),
            pltpu.VMEM((bpw, 128), _f32),
            pltpu.VMEM((bpw,), _f32),
            pltpu.VMEM((bpw,), _f32),
            pltpu.SemaphoreType.DMA,
            pltpu.SemaphoreType.DMA,
            pltpu.SemaphoreType.DMA,
        ],
    )
    def score(ufin_hbm, ifin_hbm, user_hbm, pos_hbm, neg_hbm,
              pos_out, neg_out,
              uidx, pidx, nidx, urows, prows, nrows, psc, nsc,
              sem_u, sem_p, sem_n):
        wid = lax.axis_index("s") * nc + lax.axis_index("c")
        base = wid * bpw
        pltpu.sync_copy(user_hbm.at[pl.ds(base, bpw)], uidx)
        pltpu.sync_copy(pos_hbm.at[pl.ds(base, bpw)], pidx)
        pltpu.sync_copy(neg_hbm.at[pl.ds(base, bpw)], nidx)
        cu = pltpu.async_copy(ufin_hbm.at[uidx], urows, sem_u)
        cp = pltpu.async_copy(ifin_hbm.at[pidx], prows, sem_p)
        cn = pltpu.async_copy(ifin_hbm.at[nidx], nrows, sem_n)
        cu.wait()
        cp.wait()
        cn.wait()

        lanes = lax.iota(jnp.int32, 16)

        def dot16(a, b):
            s = a * b
            for sh in (8, 4, 2, 1):
                perm = jnp.bitwise_xor(lanes, sh)
                s = s + s.at[perm].get(mode="promise_in_bounds")
            return s

        def group(g, carry):
            accp = jnp.zeros((16,), _f32)
            accn = jnp.zeros((16,), _f32)
            for k in range(16):
                b = g * 16 + k
                u = urows[b, pl.ds(0, D)]
                accp = jnp.where(lanes == k, dot16(u, prows[b, pl.ds(0, D)]), accp)
                accn = jnp.where(lanes == k, dot16(u, nrows[b, pl.ds(0, D)]), accn)
            psc[pl.ds(g * 16, 16)] = accp
            nsc[pl.ds(g * 16, 16)] = accn
            return carry

        lax.fori_loop(0, bpw // 16, group, 0)
        pltpu.sync_copy(psc, pos_out.at[pl.ds(base, bpw)])
        pltpu.sync_copy(nsc, neg_out.at[pl.ds(base, bpw)])

    return score


def kernel(user, pos_item, neg_item, user_emb, item_emb, A, A_T):
    u_fin, i_fin = _propagate(user_emb, item_emb, A, A_T)
    score = _make_score_kernel()
    pos, neg = score(u_fin, i_fin,
                     user.astype(jnp.int32),
                     pos_item.astype(jnp.int32),
                     neg_item.astype(jnp.int32))
    return pos.reshape(B, 1), neg.reshape(B, 1)


# speedup vs baseline: 1.0016x; 1.0016x over previous
import functools

import jax
import jax.numpy as jnp
from jax import lax
from jax.experimental import pallas as pl
from jax.experimental.pallas import tpu as pltpu
from jax.experimental.pallas import tpu_sc as plsc

U, I, D, B = 10000, 10000, 16, 4096
BM = 400
NB = U // BM
_TC_PARAMS = pltpu.CompilerParams(vmem_limit_bytes=64 * 1024 * 1024)

_f32 = jnp.float32


def _pass1_body(at_ref, u0f_ref, i0_ref, i1_ref, u1_ref, u1t_acc):
    j = pl.program_id(0)
    at = at_ref[...]
    i1_ref[...] = jnp.dot(at, u0f_ref[...], preferred_element_type=_f32)
    part_t = lax.dot_general(i0_ref[...], at, (((0,), (0,)), ((), ())),
                             preferred_element_type=_f32)

    @pl.when(j == 0)
    def _init():
        u1t_acc[...] = part_t

    @pl.when(j > 0)
    def _acc():
        u1t_acc[...] += part_t

    @pl.when(j == NB - 1)
    def _fin():
        u1_ref[...] = u1t_acc[...].T


def _pass2_body(a_ref, i1f_ref, u1_ref, u0_ref, i0f_ref, ufin_ref, ifin_ref,
                i2t_acc):
    j = pl.program_id(0)
    a = a_ref[...]
    u1_blk = u1_ref[...]
    u2 = jnp.dot(a, i1f_ref[...], preferred_element_type=_f32)
    ufin_ref[:, 0:D] = (u0_ref[...] + u1_blk + u2) * (1.0 / 3.0)
    part_t = lax.dot_general(u1_blk, a, (((0,), (0,)), ((), ())),
                             preferred_element_type=_f32)

    @pl.when(j == 0)
    def _init():
        i2t_acc[...] = part_t

    @pl.when(j > 0)
    def _acc():
        i2t_acc[...] += part_t

    @pl.when(j == NB - 1)
    def _fin():
        ifin_ref[:, 0:D] = (i2t_acc[...].T + i0f_ref[...] + i1f_ref[...]) * (1.0 / 3.0)


def _propagate(user_emb, item_emb, A, A_T):
    blk_a = pl.BlockSpec((BM, I), lambda j: (j, 0))
    blk_row = pl.BlockSpec((BM, D), lambda j: (j, 0))
    full_u = pl.BlockSpec((U, D), lambda j: (0, 0))
    full_i = pl.BlockSpec((I, D), lambda j: (0, 0))

    i1, u1 = pl.pallas_call(
        _pass1_body,
        grid=(NB,),
        in_specs=[blk_a, full_u, blk_row],
        out_specs=[blk_row, full_u],
        out_shape=(jax.ShapeDtypeStruct((I, D), _f32),
                   jax.ShapeDtypeStruct((U, D), _f32)),
        scratch_shapes=[pltpu.VMEM((D, U), _f32)],
        compiler_params=_TC_PARAMS,
    )(A_T, user_emb, item_emb)

    blk_row_pad = pl.BlockSpec((BM, 128), lambda j: (j, 0))
    full_i_pad = pl.BlockSpec((I, 128), lambda j: (0, 0))
    u_fin, i_fin = pl.pallas_call(
        _pass2_body,
        grid=(NB,),
        in_specs=[blk_a, full_i, blk_row, blk_row, full_i],
        out_specs=[blk_row_pad, full_i_pad],
        out_shape=(jax.ShapeDtypeStruct((U, 128), _f32),
                   jax.ShapeDtypeStruct((I, 128), _f32)),
        scratch_shapes=[pltpu.VMEM((D, I), _f32)],
        compiler_params=_TC_PARAMS,
    )(A, i1, u1, user_emb, item_emb)
    return u_fin, i_fin


def _make_score_kernel():
    info = plsc.get_sparse_core_info()
    nc, ns = info.num_cores, info.num_subcores
    nw = nc * ns
    bpw = B // nw
    mesh = plsc.VectorSubcoreMesh(core_axis_name="c", subcore_axis_name="s")

    @functools.partial(
        pl.kernel, mesh=mesh,
        out_type=(jax.ShapeDtypeStruct((B,), _f32),
                  jax.ShapeDtypeStruct((B,), _f32)),
        scratch_types=[
            pltpu.VMEM((bpw,), jnp.int32),
            pltpu.VMEM((bpw,), jnp.int32),
            pltpu.VMEM((bpw,), jnp.int32),
            pltpu.VMEM((bpw, 128), _f32),
            pltpu.VMEM((bpw, 128), _f32),
            pltpu.VMEM((bpw, 128), _f32),
            pltpu.VMEM((bpw,), _f32),
            pltpu.VMEM((bpw,), _f32),
            pltpu.SemaphoreType.DMA,
            pltpu.SemaphoreType.DMA,
            pltpu.SemaphoreType.DMA,
        ],
    )
    def score(ufin_hbm, ifin_hbm, user_hbm, pos_hbm, neg_hbm,
              pos_out, neg_out,
              uidx, pidx, nidx, urows, prows, nrows, psc, nsc,
              sem_u, sem_p, sem_n):
        wid = lax.axis_index("s") * nc + lax.axis_index("c")
        base = wid * bpw
        pltpu.sync_copy(user_hbm.at[pl.ds(base, bpw)], uidx)
        pltpu.sync_copy(pos_hbm.at[pl.ds(base, bpw)], pidx)
        pltpu.sync_copy(neg_hbm.at[pl.ds(base, bpw)], nidx)
        cu = pltpu.async_copy(ufin_hbm.at[uidx], urows, sem_u)
        cp = pltpu.async_copy(ifin_hbm.at[pidx], prows, sem_p)
        cn = pltpu.async_copy(ifin_hbm.at[nidx], nrows, sem_n)
        cu.wait()
        cp.wait()
        cn.wait()

        lanes = lax.iota(jnp.int32, 16)

        def dot16(a, b):
            s = a * b
            for sh in (8, 4, 2, 1):
                perm = jnp.bitwise_xor(lanes, sh)
                s = s + s.at[perm].get(mode="promise_in_bounds")
            return s

        def group(g, carry):
            accp = jnp.zeros((16,), _f32)
            accn = jnp.zeros((16,), _f32)
            for k in range(16):
                b = g * 16 + k
                u = urows[b, pl.ds(0, D)]
                accp = jnp.where(lanes == k, dot16(u, prows[b, pl.ds(0, D)]), accp)
                accn = jnp.where(lanes == k, dot16(u, nrows[b, pl.ds(0, D)]), accn)
            psc[pl.ds(g * 16, 16)] = accp
            nsc[pl.ds(g * 16, 16)] = accn
            return carry

        lax.fori_loop(0, bpw // 16, group, 0)
        pltpu.sync_copy(psc, pos_out.at[pl.ds(base, bpw)])
        pltpu.sync_copy(nsc, neg_out.at[pl.ds(base, bpw)])

    return score


def kernel(user, pos_item, neg_item, user_emb, item_emb, A, A_T):
    u_fin, i_fin = _propagate(user_emb, item_emb, A, A_T)
    score = _make_score_kernel()
    pos, neg = score(u_fin, i_fin,
                     user.astype(jnp.int32),
                     pos_item.astype(jnp.int32),
                     neg_item.astype(jnp.int32))
    return pos.reshape(B, 1), neg.reshape(B, 1)


# speedup vs baseline: 1.0262x; 1.0245x over previous
import functools

import jax
import jax.numpy as jnp
from jax import lax
from jax.experimental import pallas as pl
from jax.experimental.pallas import tpu as pltpu
from jax.experimental.pallas import tpu_sc as plsc

U, I, D, B = 10000, 10000, 16, 4096
BM = 400
NB = U // BM
_TC_PARAMS = pltpu.CompilerParams(vmem_limit_bytes=64 * 1024 * 1024)

_f32 = jnp.float32


def _pass1_body(at_ref, u0t_ref, i0t_ref, i1_ref, u1_ref, u0r_ref, i0r_ref,
                u1t_acc):
    j = pl.program_id(0)

    @pl.when(j == 0)
    def _relayout():
        u0r_ref[...] = u0t_ref[...].T
        i0r_ref[...] = i0t_ref[...].T

    at = at_ref[...]
    i1_ref[...] = jnp.dot(at, u0r_ref[...], preferred_element_type=_f32)
    part_t = lax.dot_general(i0r_ref[pl.ds(j * BM, BM), :], at,
                             (((0,), (0,)), ((), ())),
                             preferred_element_type=_f32)

    @pl.when(j == 0)
    def _init():
        u1t_acc[...] = part_t

    @pl.when(j > 0)
    def _acc():
        u1t_acc[...] += part_t

    @pl.when(j == NB - 1)
    def _fin():
        u1_ref[...] = u1t_acc[...].T


def _pass2_body(a_ref, i1f_ref, u1_ref, u0_ref, i0f_ref, ufin_ref, ifin_ref,
                i2t_acc):
    j = pl.program_id(0)
    a = a_ref[...]
    u1_blk = u1_ref[...]
    u2 = jnp.dot(a, i1f_ref[...], preferred_element_type=_f32)
    ufin_ref[:, 0:D] = (u0_ref[...] + u1_blk + u2) * (1.0 / 3.0)
    part_t = lax.dot_general(u1_blk, a, (((0,), (0,)), ((), ())),
                             preferred_element_type=_f32)

    @pl.when(j == 0)
    def _init():
        i2t_acc[...] = part_t

    @pl.when(j > 0)
    def _acc():
        i2t_acc[...] += part_t

    @pl.when(j == NB - 1)
    def _fin():
        ifin_ref[:, 0:D] = (i2t_acc[...].T + i0f_ref[...] + i1f_ref[...]) * (1.0 / 3.0)


def _propagate(user_emb, item_emb, A, A_T):
    blk_a = pl.BlockSpec((BM, I), lambda j: (j, 0))
    blk_row = pl.BlockSpec((BM, D), lambda j: (j, 0))
    full_u = pl.BlockSpec((U, D), lambda j: (0, 0))
    full_i = pl.BlockSpec((I, D), lambda j: (0, 0))

    full_ut = pl.BlockSpec((D, U), lambda j: (0, 0))
    full_it = pl.BlockSpec((D, I), lambda j: (0, 0))
    i1, u1, u0r, i0r = pl.pallas_call(
        _pass1_body,
        grid=(NB,),
        in_specs=[blk_a, full_ut, full_it],
        out_specs=[blk_row, full_u, full_u, full_i],
        out_shape=(jax.ShapeDtypeStruct((I, D), _f32),
                   jax.ShapeDtypeStruct((U, D), _f32),
                   jax.ShapeDtypeStruct((U, D), _f32),
                   jax.ShapeDtypeStruct((I, D), _f32)),
        scratch_shapes=[pltpu.VMEM((D, U), _f32)],
        compiler_params=_TC_PARAMS,
    )(A_T, user_emb.T, item_emb.T)

    blk_row_pad = pl.BlockSpec((BM, 128), lambda j: (j, 0))
    full_i_pad = pl.BlockSpec((I, 128), lambda j: (0, 0))
    u_fin, i_fin = pl.pallas_call(
        _pass2_body,
        grid=(NB,),
        in_specs=[blk_a, full_i, blk_row, blk_row, full_i],
        out_specs=[blk_row_pad, full_i_pad],
        out_shape=(jax.ShapeDtypeStruct((U, 128), _f32),
                   jax.ShapeDtypeStruct((I, 128), _f32)),
        scratch_shapes=[pltpu.VMEM((D, I), _f32)],
        compiler_params=_TC_PARAMS,
    )(A, i1, u1, u0r, i0r)
    return u_fin, i_fin


def _make_score_kernel():
    info = plsc.get_sparse_core_info()
    nc, ns = info.num_cores, info.num_subcores
    nw = nc * ns
    bpw = B // nw
    mesh = plsc.VectorSubcoreMesh(core_axis_name="c", subcore_axis_name="s")

    @functools.partial(
        pl.kernel, mesh=mesh,
        out_type=(jax.ShapeDtypeStruct((B,), _f32),
                  jax.ShapeDtypeStruct((B,), _f32)),
        scratch_types=[
            pltpu.VMEM((bpw,), jnp.int32),
            pltpu.VMEM((bpw,), jnp.int32),
            pltpu.VMEM((bpw,), jnp.int32),
            pltpu.VMEM((bpw, 128), _f32),
            pltpu.VMEM((bpw, 128), _f32),
            pltpu.VMEM((bpw, 128), _f32),
            pltpu.VMEM((bpw,), _f32),
            pltpu.VMEM((bpw,), _f32),
            pltpu.SemaphoreType.DMA,
            pltpu.SemaphoreType.DMA,
            pltpu.SemaphoreType.DMA,
        ],
    )
    def score(ufin_hbm, ifin_hbm, user_hbm, pos_hbm, neg_hbm,
              pos_out, neg_out,
              uidx, pidx, nidx, urows, prows, nrows, psc, nsc,
              sem_u, sem_p, sem_n):
        wid = lax.axis_index("s") * nc + lax.axis_index("c")
        base = wid * bpw
        pltpu.sync_copy(user_hbm.at[pl.ds(base, bpw)], uidx)
        pltpu.sync_copy(pos_hbm.at[pl.ds(base, bpw)], pidx)
        pltpu.sync_copy(neg_hbm.at[pl.ds(base, bpw)], nidx)
        cu = pltpu.async_copy(ufin_hbm.at[uidx], urows, sem_u)
        cp = pltpu.async_copy(ifin_hbm.at[pidx], prows, sem_p)
        cn = pltpu.async_copy(ifin_hbm.at[nidx], nrows, sem_n)
        cu.wait()
        cp.wait()
        cn.wait()

        lanes = lax.iota(jnp.int32, 16)

        def dot16(a, b):
            s = a * b
            for sh in (8, 4, 2, 1):
                perm = jnp.bitwise_xor(lanes, sh)
                s = s + s.at[perm].get(mode="promise_in_bounds")
            return s

        def group(g, carry):
            accp = jnp.zeros((16,), _f32)
            accn = jnp.zeros((16,), _f32)
            for k in range(16):
                b = g * 16 + k
                u = urows[b, pl.ds(0, D)]
                accp = jnp.where(lanes == k, dot16(u, prows[b, pl.ds(0, D)]), accp)
                accn = jnp.where(lanes == k, dot16(u, nrows[b, pl.ds(0, D)]), accn)
            psc[pl.ds(g * 16, 16)] = accp
            nsc[pl.ds(g * 16, 16)] = accn
            return carry

        lax.fori_loop(0, bpw // 16, group, 0)
        pltpu.sync_copy(psc, pos_out.at[pl.ds(base, bpw)])
        pltpu.sync_copy(nsc, neg_out.at[pl.ds(base, bpw)])

    return score


def kernel(user, pos_item, neg_item, user_emb, item_emb, A, A_T):
    u_fin, i_fin = _propagate(user_emb, item_emb, A, A_T)
    score = _make_score_kernel()
    pos, neg = score(u_fin, i_fin,
                     user.astype(jnp.int32),
                     pos_item.astype(jnp.int32),
                     neg_item.astype(jnp.int32))
    return pos.reshape(B, 1), neg.reshape(B, 1)
